# baseline (speedup 1.0000x reference)
import functools

import jax
import jax.numpy as jnp
from jax import lax
from jax.experimental import pallas as pl
from jax.experimental.pallas import tpu as pltpu
from jax.experimental.pallas import tpu_sc as plsc

SEQ_LEN = 2048
EMBED_DIM = 64
BATCH = 4
VOCAB = 1000000
B_TOT = BATCH * SEQ_LEN

NUM_CORES = 2
NUM_SUBCORES = 16
LANES = 16
NW = NUM_CORES * NUM_SUBCORES
BPW = B_TOT // NW
OPW = BPW // 2

_mesh = plsc.VectorSubcoreMesh(core_axis_name="c", subcore_axis_name="s")


@functools.partial(
    pl.kernel,
    mesh=_mesh,
    out_type=jax.ShapeDtypeStruct((B_TOT // 2, 2 * EMBED_DIM), jnp.float32),
    scratch_types=[
        pltpu.VMEM((BPW,), jnp.int32),
        pltpu.VMEM((BPW,), jnp.int32),
        pltpu.VMEM((BPW, 2 * EMBED_DIM), jnp.float32),
        pltpu.VMEM((OPW, 2 * EMBED_DIM), jnp.float32),
        pltpu.VMEM((OPW, 2 * EMBED_DIM), jnp.float32),
        pltpu.SemaphoreType.DMA,
    ],
)
def _embed_sc(idx_hbm, tok_hbm, pos_hbm, out_hbm, idx_v, idx2_v, rows_v,
              pos_v, out_v, sem):
    wid = lax.axis_index("s") * NUM_CORES + lax.axis_index("c")
    base = wid * BPW
    pos_base = lax.rem(wid * OPW, SEQ_LEN // 2)

    pltpu.sync_copy(idx_hbm.at[pl.ds(base, BPW)], idx_v)

    def _halve(i, carry):
        sl = pl.ds(i * LANES, LANES)
        idx2_v[sl] = lax.shift_right_logical(idx_v[sl], 1)
        return carry

    lax.fori_loop(0, BPW // LANES, _halve, 0)

    gather = pltpu.async_copy(tok_hbm.at[idx2_v], rows_v, sem)
    pltpu.sync_copy(pos_hbm.at[pl.ds(pos_base, OPW)], pos_v)
    gather.wait()

    def _assemble(g, carry):
        iv = idx_v[pl.ds(g * LANES, LANES)]
        for l in range(LANES):
            r = g * LANES + l
            j = g * (LANES // 2) + l // 2
            src = lax.rem(iv[l], 2) * EMBED_DIM
            dst = (l % 2) * EMBED_DIM
            for c in range(EMBED_DIM // LANES):
                d = pl.ds(dst + c * LANES, LANES)
                out_v[j, d] = (rows_v[r, pl.ds(src + c * LANES, LANES)]
                               + pos_v[j, d])
        return carry

    lax.fori_loop(0, BPW // LANES, _assemble, 0)

    pltpu.sync_copy(out_v, out_hbm.at[pl.ds(wid * OPW, OPW)])


def kernel(inputs, token_table, pos_table):
    flat = inputs.reshape(B_TOT)
    tok2 = token_table.reshape(VOCAB // 2, 2 * EMBED_DIM)
    pos2 = pos_table.reshape(SEQ_LEN // 2, 2 * EMBED_DIM)
    out = _embed_sc(flat, tok2, pos2)
    return out.reshape(BATCH, SEQ_LEN, EMBED_DIM)


# speedup vs baseline: 5.2302x; 5.2302x over previous
import functools

import jax
import jax.numpy as jnp
from jax import lax
from jax.experimental import pallas as pl
from jax.experimental.pallas import tpu as pltpu
from jax.experimental.pallas import tpu_sc as plsc

SEQ_LEN = 2048
EMBED_DIM = 64
BATCH = 4
VOCAB = 1000000
B_TOT = BATCH * SEQ_LEN

NUM_CORES = 2
NUM_SUBCORES = 16
LANES = 16
NW = NUM_CORES * NUM_SUBCORES
BPW = B_TOT // NW
RING = 8
NGRP = BPW // RING

_mesh = plsc.VectorSubcoreMesh(core_axis_name="c", subcore_axis_name="s")


@functools.partial(
    pl.kernel,
    mesh=_mesh,
    out_type=jax.ShapeDtypeStruct((B_TOT * EMBED_DIM,), jnp.float32),
    scratch_types=[
        pltpu.VMEM((BPW + LANES,), jnp.int32),
        pltpu.VMEM((RING, EMBED_DIM, 128), jnp.float32),
        pltpu.VMEM((BPW * EMBED_DIM,), jnp.float32),
        pltpu.VMEM((EMBED_DIM, BPW), jnp.float32),
        [pltpu.SemaphoreType.DMA] * RING,
        pltpu.SemaphoreType.DMA,
    ],
    compiler_params=pltpu.CompilerParams(needs_layout_passes=False),
)
def _embed_sc(idx_hbm, tokt_hbm, post_hbm, out_hbm, idx_v, blk_v, g_v,
              pos_v, bsems, psem):
    wid = lax.axis_index("s") * NUM_CORES + lax.axis_index("c")
    base = wid * BPW
    pos_base = lax.rem(base, SEQ_LEN)

    pltpu.sync_copy(idx_hbm.at[pl.ds(base, BPW)], idx_v.at[pl.ds(0, BPW)])
    pos_copy = pltpu.async_copy(
        post_hbm.at[:, pl.ds(pos_base, BPW)], pos_v, psem)

    def _fire(r, v):
        vb = pl.multiple_of((v >> 7) << 7, 128)
        pltpu.async_copy(
            tokt_hbm.at[:, pl.ds(vb, 128)], blk_v.at[r], bsems[r])

    iv0 = idx_v[pl.ds(0, LANES)]
    for r in range(RING):
        _fire(r, iv0[r])
    pos_copy.wait()

    rows = [lax.iota(jnp.int32, LANES) + c * LANES
            for c in range(EMBED_DIM // LANES)]

    def _group(g, carry):
        ivc = idx_v[pl.ds(g * RING, LANES)]
        ivn = idx_v[pl.ds(g * RING + RING, LANES)]
        for r in range(RING):
            pltpu.make_async_copy(
                tokt_hbm.at[:, pl.ds(0, 128)], blk_v.at[r], bsems[r]).wait()
            j = g * RING + r
            ln = jnp.full((LANES,), ivc[r] & 127, dtype=jnp.int32)
            pj = jnp.full((LANES,), j, dtype=jnp.int32)
            for c in range(EMBED_DIM // LANES):
                tok = plsc.load_gather(blk_v.at[r], [rows[c], ln])
                pos = plsc.load_gather(pos_v, [rows[c], pj])
                g_v[pl.ds(j * EMBED_DIM + c * LANES, LANES)] = tok + pos

            @pl.when(g < NGRP - 1)
            def _():
                _fire(r, ivn[r])

        return carry

    lax.fori_loop(0, NGRP, _group, 0)

    pltpu.sync_copy(g_v, out_hbm.at[pl.ds(base * EMBED_DIM, BPW * EMBED_DIM)])


def kernel(inputs, token_table, pos_table):
    flat = inputs.reshape(B_TOT)
    out = _embed_sc(flat, token_table.T, pos_table.T)
    return out.reshape(BATCH, SEQ_LEN, EMBED_DIM)


# speedup vs baseline: 5.4569x; 1.0433x over previous
import functools

import jax
import jax.numpy as jnp
from jax import lax
from jax.experimental import pallas as pl
from jax.experimental.pallas import tpu as pltpu
from jax.experimental.pallas import tpu_sc as plsc

SEQ_LEN = 2048
EMBED_DIM = 64
BATCH = 4
VOCAB = 1000000
B_TOT = BATCH * SEQ_LEN

NUM_CORES = 2
NUM_SUBCORES = 16
LANES = 16
NW = NUM_CORES * NUM_SUBCORES
BPW = B_TOT // NW
RING = 8
NGRP = BPW // RING

_mesh = plsc.VectorSubcoreMesh(core_axis_name="c", subcore_axis_name="s")


@functools.partial(
    pl.kernel,
    mesh=_mesh,
    out_type=jax.ShapeDtypeStruct((BATCH, EMBED_DIM, SEQ_LEN), jnp.float32),
    scratch_types=[
        pltpu.VMEM((BPW + LANES,), jnp.int32),
        pltpu.VMEM((RING, EMBED_DIM, 128), jnp.float32),
        pltpu.VMEM((EMBED_DIM, BPW), jnp.float32),
        [pltpu.SemaphoreType.DMA] * RING,
        pltpu.SemaphoreType.DMA,
    ],
    compiler_params=pltpu.CompilerParams(needs_layout_passes=False),
)
def _embed_sc(idx_hbm, tokt_hbm, post_hbm, out_hbm, idx_v, blk_v, slab_v,
              bsems, psem):
    wid = lax.axis_index("s") * NUM_CORES + lax.axis_index("c")
    base = wid * BPW
    batch = base // SEQ_LEN
    seq0 = lax.rem(base, SEQ_LEN)

    pltpu.sync_copy(idx_hbm.at[pl.ds(base, BPW)], idx_v.at[pl.ds(0, BPW)])
    pos_copy = pltpu.async_copy(
        post_hbm.at[:, pl.ds(seq0, BPW)], slab_v, psem)

    def _fire(r, v):
        vb = pl.multiple_of((v >> 7) << 7, 128)
        pltpu.async_copy(
            tokt_hbm.at[:, pl.ds(vb, 128)], blk_v.at[r], bsems[r])

    iv0 = idx_v[pl.ds(0, LANES)]
    for r in range(RING):
        _fire(r, iv0[r])
    pos_copy.wait()

    rows = [lax.iota(jnp.int32, LANES) + c * LANES
            for c in range(EMBED_DIM // LANES)]

    def _group(g, carry):
        ivc = idx_v[pl.ds(g * RING, LANES)]
        ivn = idx_v[pl.ds(g * RING + RING, LANES)]
        for r in range(RING):
            pltpu.make_async_copy(
                tokt_hbm.at[:, pl.ds(0, 128)], blk_v.at[r], bsems[r]).wait()
            j = g * RING + r
            ln = jnp.full((LANES,), ivc[r] & 127, dtype=jnp.int32)
            pj = jnp.full((LANES,), j, dtype=jnp.int32)
            for c in range(EMBED_DIM // LANES):
                tok = plsc.load_gather(blk_v.at[r], [rows[c], ln])
                plsc.addupdate_scatter(slab_v, [rows[c], pj], tok)

            @pl.when(g < NGRP - 1)
            def _():
                _fire(r, ivn[r])

        return carry

    lax.fori_loop(0, NGRP, _group, 0)

    pltpu.sync_copy(slab_v, out_hbm.at[batch, :, pl.ds(seq0, BPW)])


def kernel(inputs, token_table, pos_table):
    flat = inputs.reshape(B_TOT)
    out = _embed_sc(flat, token_table.T, pos_table.T)
    return out.transpose(0, 2, 1)
